# Initial kernel scaffold; baseline (speedup 1.0000x reference)
import jax
import jax.numpy as jnp
from jax.experimental import pallas as pl


def kernel(net, mask):
    raise NotImplementedError("write your pallas kernel here")


# speedup vs baseline: 4.8949x; 4.8949x over previous
import jax
import jax.numpy as jnp
from jax import lax
from jax.experimental import pallas as pl
from jax.experimental.pallas import tpu as pltpu
from jax.experimental.pallas import tpu_sc as plsc

B = 8
HW = 112 * 112
P = 224 * 224
C = 96
NCB = 6
CB = 16
NC = 2
NS = 16
ROWS = HW // NS
PROWS = P // NS
B_PER_CORE = B // NC


def _body(*refs):
  nets = refs[0:NCB]
  masks = refs[NCB:2 * NCB]
  zeros_ref = refs[2 * NCB]
  out_ref = refs[2 * NCB + 1]
  accum, mask_v, vals_v, idx_v, zero_v = refs[2 * NCB + 2:]

  cid = lax.axis_index("c")
  sid = lax.axis_index("s")
  lane = lax.iota(jnp.int32, 16)

  pltpu.sync_copy(zeros_ref, zero_v)

  for cb_i in range(NCB):
    net_cb = nets[cb_i]
    mask_cb = masks[cb_i]

    def task_body(bt, _, net_cb=net_cb, mask_cb=mask_cb, cb_i=cb_i):
      b = cid * B_PER_CORE + bt

      base = sid * PROWS * CB
      for q in range(4):
        pltpu.sync_copy(zero_v, accum.at[pl.ds(base + q * HW, HW)])

      r0 = sid * ROWS
      pltpu.sync_copy(mask_cb.at[b, pl.ds(r0, ROWS), pl.ds(0, CB)], mask_v)
      pltpu.sync_copy(net_cb.at[b, pl.ds(r0 * CB, HW)], vals_v)

      def decode(j, _):
        m = mask_v[j]
        x = m >> 5
        a = x >> 10
        r = x & 1023
        p = a * 341 + (((a + r) * 683) >> 11)
        idx_v[pl.ds(j * 16, 16)] = p * CB + lane
        return 0
      lax.fori_loop(0, ROWS, decode, 0)

      plsc.subcore_barrier()

      pltpu.sync_copy(vals_v, accum.at[idx_v], add=True)

      plsc.subcore_barrier()

      pltpu.sync_copy(
          accum.at[pl.ds(sid * PROWS * CB, PROWS * CB)],
          out_ref.at[b, cb_i, pl.ds(sid * PROWS * CB, PROWS * CB)])
      return 0

    lax.fori_loop(0, B_PER_CORE, task_body, 0)


@jax.jit
def kernel(net, mask):
  net3 = net.reshape(B, HW, C)
  mask3 = mask.reshape(B, HW, C)
  nets = [net3[:, :, k * CB:(k + 1) * CB].reshape(B, HW * CB)
          for k in range(NCB)]
  masks = [mask3[:, :, k * CB:(k + 1) * CB] for k in range(NCB)]
  zeros = jnp.zeros((HW,), jnp.float32)
  mesh = plsc.VectorSubcoreMesh(
      core_axis_name="c", subcore_axis_name="s", num_cores=NC, num_subcores=NS)
  f = pl.kernel(
      _body,
      out_type=jax.ShapeDtypeStruct((B, NCB, P * CB), jnp.float32),
      mesh=mesh,
      compiler_params=pltpu.CompilerParams(use_tc_tiling_on_sc=False),
      scratch_types=[
          pltpu.VMEM_SHARED((P * CB,), jnp.float32),
          pltpu.VMEM((ROWS, CB), jnp.int32),
          pltpu.VMEM((HW,), jnp.float32),
          pltpu.VMEM((HW,), jnp.int32),
          pltpu.VMEM((HW,), jnp.float32),
      ],
  )
  out = f(*nets, *masks, zeros)
  return (out.reshape(B, NCB, P, CB)
          .transpose(0, 2, 1, 3)
          .reshape(B, 224, 224, C))


# speedup vs baseline: 5.5035x; 1.1243x over previous
import jax
import jax.numpy as jnp
from jax import lax
from jax.experimental import pallas as pl
from jax.experimental.pallas import tpu as pltpu
from jax.experimental.pallas import tpu_sc as plsc

B = 8
HW = 112 * 112
P = 224 * 224
C = 96
NCB = 6
CB = 16
NC = 2
NS = 16
ROWS = HW // NS
PROWS = P // NS
TASKS_PER_CORE = (B * NCB) // NC


def _body(net_ref, mask_ref, zeros_ref, out_ref,
          accum, mask_v, vals2_v, vals_v, idx_v, zero_v):
  cid = lax.axis_index("c")
  sid = lax.axis_index("s")
  lane = lax.iota(jnp.int32, 16)

  pltpu.sync_copy(zeros_ref, zero_v)

  def task_body(t, _):
    task = cid * TASKS_PER_CORE + t
    b = task // NCB
    cb = (task % NCB) * CB

    base = sid * PROWS * CB
    for q in range(4):
      pltpu.sync_copy(zero_v, accum.at[pl.ds(base + q * HW, HW)])

    r0 = sid * ROWS
    pltpu.sync_copy(mask_ref.at[b, pl.ds(r0, ROWS), pl.ds(cb, CB)], mask_v)
    pltpu.sync_copy(net_ref.at[b, pl.ds(r0, ROWS), pl.ds(cb, CB)], vals2_v)

    def decode(j, _):
      m = mask_v[j]
      x = m >> 5
      a = x >> 10
      r = x & 1023
      p = a * 341 + (((a + r) * 683) >> 11)
      idx_v[pl.ds(j * 16, 16)] = p * CB + lane
      vals_v[pl.ds(j * 16, 16)] = vals2_v[j]
      return 0
    lax.fori_loop(0, ROWS, decode, 0)

    plsc.subcore_barrier()

    pltpu.sync_copy(vals_v, accum.at[idx_v], add=True)

    plsc.subcore_barrier()

    pltpu.sync_copy(
        accum.at[pl.ds(sid * PROWS * CB, PROWS * CB)],
        out_ref.at[b, task % NCB, pl.ds(sid * PROWS * CB, PROWS * CB)])
    return 0

  lax.fori_loop(0, TASKS_PER_CORE, task_body, 0)


@jax.jit
def kernel(net, mask):
  net3 = net.reshape(B, HW, C)
  mask3 = mask.reshape(B, HW, C)
  zeros = jnp.zeros((HW,), jnp.float32)
  mesh = plsc.VectorSubcoreMesh(
      core_axis_name="c", subcore_axis_name="s", num_cores=NC, num_subcores=NS)
  f = pl.kernel(
      _body,
      out_type=jax.ShapeDtypeStruct((B, NCB, P * CB), jnp.float32),
      mesh=mesh,
      compiler_params=pltpu.CompilerParams(use_tc_tiling_on_sc=False),
      scratch_types=[
          pltpu.VMEM_SHARED((P * CB,), jnp.float32),
          pltpu.VMEM((ROWS, CB), jnp.int32),
          pltpu.VMEM((ROWS, CB), jnp.float32),
          pltpu.VMEM((HW,), jnp.float32),
          pltpu.VMEM((HW,), jnp.int32),
          pltpu.VMEM((HW,), jnp.float32),
      ],
  )
  out = f(net3, mask3, zeros)
  return (out.reshape(B, NCB, P, CB)
          .transpose(0, 2, 1, 3)
          .reshape(B, 224, 224, C))


# speedup vs baseline: 24.4533x; 4.4432x over previous
import jax
import jax.numpy as jnp
from jax import lax
from jax.experimental import pallas as pl
from jax.experimental.pallas import tpu as pltpu
from jax.experimental.pallas import tpu_sc as plsc

B = 8
HW = 112 * 112
P = 224 * 224
C = 96
NCB = 6
CB = 16
NC = 2
NS = 16
ROWS = HW // NS
PROWS = P // NS
TASKS_PER_CORE = (B * NCB) // NC


def _body(net_ref, mask_ref, out_ref,
          accum, mask_v, vals2_v, vals_v, idx_v, zero_v):
  cid = lax.axis_index("c")
  sid = lax.axis_index("s")
  lane = lax.iota(jnp.int32, 16)
  zf16 = jnp.zeros((16,), jnp.float32)

  def _zinit(j, _):
    zero_v[pl.ds(j * 16, 16)] = zf16
    return 0
  lax.fori_loop(0, ROWS, _zinit, 0)

  def task_body(t, _):
    task = cid * TASKS_PER_CORE + t
    b = task // NCB
    cb = (task % NCB) * CB

    base = sid * PROWS * CB
    for q in range(4):
      pltpu.sync_copy(zero_v, accum.at[pl.ds(base + q * HW, HW)])

    r0 = sid * ROWS
    pltpu.sync_copy(mask_ref.at[b, pl.ds(r0, ROWS), pl.ds(cb, CB)], mask_v)
    pltpu.sync_copy(net_ref.at[b, pl.ds(r0, ROWS), pl.ds(cb, CB)], vals2_v)

    def decode(j, _):
      m = mask_v[j]
      x = m >> 5
      a = x >> 10
      r = x & 1023
      p = a * 341 + (((a + r) * 683) >> 11)
      idx_v[pl.ds(j * 16, 16)] = p * CB + lane
      vals_v[pl.ds(j * 16, 16)] = vals2_v[j]
      return 0
    lax.fori_loop(0, ROWS, decode, 0)

    plsc.subcore_barrier()

    pltpu.sync_copy(vals_v, accum.at[idx_v], add=True)

    plsc.subcore_barrier()

    for q in range(4):
      pltpu.sync_copy(accum.at[pl.ds(base + q * HW, HW)], vals_v)

      def review(j, _):
        vals2_v[j] = vals_v[pl.ds(j * 16, 16)]
        return 0
      lax.fori_loop(0, ROWS, review, 0)

      pltpu.sync_copy(
          vals2_v,
          out_ref.at[b, pl.ds(sid * PROWS + q * ROWS, ROWS), pl.ds(cb, CB)])
    return 0

  lax.fori_loop(0, TASKS_PER_CORE, task_body, 0)


@jax.jit
def kernel(net, mask):
  net3 = net.reshape(B, HW, C)
  mask3 = mask.reshape(B, HW, C)
  mesh = plsc.VectorSubcoreMesh(
      core_axis_name="c", subcore_axis_name="s", num_cores=NC, num_subcores=NS)
  f = pl.kernel(
      _body,
      out_type=jax.ShapeDtypeStruct((B, P, C), jnp.float32),
      mesh=mesh,
      compiler_params=pltpu.CompilerParams(use_tc_tiling_on_sc=False),
      scratch_types=[
          pltpu.VMEM_SHARED((P * CB,), jnp.float32),
          pltpu.VMEM((ROWS, CB), jnp.int32),
          pltpu.VMEM((ROWS, CB), jnp.float32),
          pltpu.VMEM((HW,), jnp.float32),
          pltpu.VMEM((HW,), jnp.int32),
          pltpu.VMEM((HW,), jnp.float32),
      ],
  )
  out = f(net3, mask3)
  return out.reshape(B, 224, 224, C)
